# Initial kernel scaffold; baseline (speedup 1.0000x reference)
import functools

import jax
import jax.numpy as jnp
from jax import lax
from jax.experimental import pallas as pl
from jax.experimental.pallas import tpu as pltpu
from jax.experimental.pallas import tpu_sc as plsc

NC = 2
NS = 16
NW = NC * NS

CHUNK = 128


def _zero_fill_2d(buf, rows, cols):
    zero = jnp.zeros((16,), jnp.float32)

    def body(i, carry):
        for j in range(cols // 16):
            buf[i, pl.ds(j * 16, 16)] = zero
        return carry

    lax.fori_loop(0, rows, body, 0)


def _fill_1d(buf, n, value):
    vec = jnp.full((16,), value, jnp.float32)
    for j in range(n // 16):
        buf[pl.ds(j * 16, 16)] = vec


def _make_deg_kernel(n_pad, nstream, max_streams):
    mesh = plsc.VectorSubcoreMesh(core_axis_name="c", subcore_axis_name="s",
                                  num_cores=NC, num_subcores=NS)
    stripe = n_pad // NS

    @functools.partial(
        pl.kernel,
        out_type=jax.ShapeDtypeStruct((NC, 2, n_pad), jnp.float32),
        mesh=mesh,
        scratch_types=[
            pltpu.VMEM((max_streams, CHUNK), jnp.int32),
            pltpu.VMEM((max_streams, CHUNK), jnp.int32),
            pltpu.VMEM((CHUNK,), jnp.float32),
            pltpu.VMEM((stripe,), jnp.float32),
            pltpu.VMEM_SHARED((n_pad,), jnp.float32),
            pltpu.VMEM_SHARED((n_pad,), jnp.float32),
        ],
    )
    def deg_kernel(src_hbm, dst_hbm, deg_out, srcidx, dstidx, ones_v, zstripe,
                   outdeg_sh, indeg_sh):
        c = lax.axis_index("c")
        s = lax.axis_index("s")
        w = c * NS + s

        _fill_1d(zstripe, stripe, 0.0)
        _fill_1d(ones_v, CHUNK, 1.0)
        pltpu.sync_copy(zstripe, outdeg_sh.at[pl.ds(s * stripe, stripe)])
        pltpu.sync_copy(zstripe, indeg_sh.at[pl.ds(s * stripe, stripe)])
        plsc.subcore_barrier()

        lo = (w * nstream) // NW
        hi = ((w + 1) * nstream) // NW
        pltpu.sync_copy(src_hbm.at[pl.ds(lo, max_streams)], srcidx)
        pltpu.sync_copy(dst_hbm.at[pl.ds(lo, max_streams)], dstidx)

        def body(j, carry):
            pltpu.sync_copy(ones_v, outdeg_sh.at[srcidx.at[j]], add=True)
            pltpu.sync_copy(ones_v, indeg_sh.at[dstidx.at[j]], add=True)
            return carry

        lax.fori_loop(0, hi - lo, body, 0)
        plsc.subcore_barrier()

        pltpu.sync_copy(outdeg_sh.at[pl.ds(s * stripe, stripe)],
                        deg_out.at[c, 0, pl.ds(s * stripe, stripe)])
        pltpu.sync_copy(indeg_sh.at[pl.ds(s * stripe, stripe)],
                        deg_out.at[c, 1, pl.ds(s * stripe, stripe)])

    return deg_kernel


def _make_scatter_kernel(n, d, nstream, max_streams):
    mesh = plsc.VectorSubcoreMesh(core_axis_name="c", subcore_axis_name="s",
                                  num_cores=NC, num_subcores=NS)
    stripe = n // NS
    zrows = 125
    assert stripe % zrows == 0

    @functools.partial(
        pl.kernel,
        out_type=jax.ShapeDtypeStruct((NC, n, d), jnp.float32),
        mesh=mesh,
        scratch_types=[
            pltpu.VMEM((max_streams, CHUNK), jnp.int32),
            pltpu.VMEM((max_streams, CHUNK), jnp.int32),
            pltpu.VMEM((CHUNK, d), jnp.float32),
            pltpu.VMEM((zrows, d), jnp.float32),
            pltpu.VMEM_SHARED((n, d), jnp.float32),
            pltpu.SemaphoreType.DMA,
        ],
    )
    def scatter_kernel(src_hbm, dst_hbm, feat_hbm, s_out, srcidx, dstidx,
                       rows, zblock, s_sh, gsem):
        c = lax.axis_index("c")
        s = lax.axis_index("s")
        w = c * NS + s

        _zero_fill_2d(zblock, zrows, d)
        for k in range(stripe // zrows):
            pltpu.sync_copy(
                zblock, s_sh.at[pl.ds(s * stripe + k * zrows, zrows)])
        plsc.subcore_barrier()

        lo = (w * nstream) // NW
        hi = ((w + 1) * nstream) // NW
        pltpu.sync_copy(src_hbm.at[pl.ds(lo, max_streams)], srcidx)
        pltpu.sync_copy(dst_hbm.at[pl.ds(lo, max_streams)], dstidx)

        def body(j, carry):
            pltpu.async_copy(feat_hbm.at[srcidx.at[j]], rows, gsem).wait()
            pltpu.sync_copy(rows, s_sh.at[dstidx.at[j]], add=True)
            return carry

        lax.fori_loop(0, hi - lo, body, 0)
        plsc.subcore_barrier()

        pltpu.sync_copy(s_sh.at[pl.ds(s * stripe, stripe)],
                        s_out.at[c, pl.ds(s * stripe, stripe)])

    return scatter_kernel


def _prescale_body(feat_ref, deg_ref, out_ref, norm_ref):
    od = deg_ref[0, 0, :] + deg_ref[1, 0, :]
    out_ref[...] = feat_ref[...] * lax.rsqrt(jnp.maximum(od, 1.0))[:, None]
    idg = deg_ref[0, 1, :] + deg_ref[1, 1, :]
    norm_ref[...] = lax.rsqrt(jnp.maximum(idg, 1.0))[:, None]


def _epilogue_body(sp_ref, feat_ref, x_ref, norm_ref, ws_ref, wi_ref, out_ref):
    s_sum = sp_ref[0] + sp_ref[1]
    r = jnp.dot(s_sum, ws_ref[...], preferred_element_type=jnp.float32)
    r = r + jnp.dot(feat_ref[...] * s_sum, wi_ref[...],
                    preferred_element_type=jnp.float32)
    out_ref[...] = (r * norm_ref[...] + x_ref[...]) * 0.5


def kernel(feature, edge_index, weight_self, weight_interaction):
    n, d = feature.shape
    e = edge_index.shape[1]
    nstream = e // CHUNK
    max_streams = -(-nstream // NW)
    n_pad = NS * (-(-(n // NS) // 8) * 8)

    src2d = edge_index[0].reshape(nstream, CHUNK)
    dst2d = edge_index[1].reshape(nstream, CHUNK)

    deg = _make_deg_kernel(n_pad, nstream, max_streams)(src2d, dst2d)
    deg = deg[:, :, :n]

    feat, norm_in = pl.pallas_call(
        _prescale_body,
        out_shape=(jax.ShapeDtypeStruct((n, d), jnp.float32),
                   jax.ShapeDtypeStruct((n, 1), jnp.float32)),
    )(feature, deg)

    s_part = _make_scatter_kernel(n, d, nstream, max_streams)(
        src2d, dst2d, feat)

    blk = 1000
    grid = n // blk
    rst = pl.pallas_call(
        _epilogue_body,
        grid=(grid,),
        in_specs=[
            pl.BlockSpec((NC, blk, d), lambda i: (0, i, 0)),
            pl.BlockSpec((blk, d), lambda i: (i, 0)),
            pl.BlockSpec((blk, d), lambda i: (i, 0)),
            pl.BlockSpec((blk, 1), lambda i: (i, 0)),
            pl.BlockSpec((d, d), lambda i: (0, 0)),
            pl.BlockSpec((d, d), lambda i: (0, 0)),
        ],
        out_specs=pl.BlockSpec((blk, d), lambda i: (i, 0)),
        out_shape=jax.ShapeDtypeStruct((n, d), jnp.float32),
    )(s_part, feat, feature, norm_in, weight_self, weight_interaction)

    return rst


# speedup vs baseline: 9.5563x; 9.5563x over previous
import functools

import jax
import jax.numpy as jnp
from jax import lax
from jax.experimental import pallas as pl
from jax.experimental.pallas import tpu as pltpu
from jax.experimental.pallas import tpu_sc as plsc

NC = 2
NS = 16
NW = NC * NS

CHUNK = 128


def _zero_fill_2d(buf, rows, cols):
    zero = jnp.zeros((16,), jnp.float32)

    def body(i, carry):
        for j in range(cols // 16):
            buf[i, pl.ds(j * 16, 16)] = zero
        return carry

    lax.fori_loop(0, rows, body, 0)


def _fill_1d(buf, n, value):
    vec = jnp.full((16,), value, jnp.float32)
    for j in range(n // 16):
        buf[pl.ds(j * 16, 16)] = vec


def _make_deg_kernel(n_tab, spw):
    mesh = plsc.VectorSubcoreMesh(core_axis_name="c", subcore_axis_name="s",
                                  num_cores=NC, num_subcores=NS)
    stripe = n_tab // NS

    @functools.partial(
        pl.kernel,
        out_type=jax.ShapeDtypeStruct((NC, 2, n_tab), jnp.float32),
        mesh=mesh,
        scratch_types=[
            pltpu.VMEM((spw, CHUNK), jnp.int32),
            pltpu.VMEM((spw, CHUNK), jnp.int32),
            pltpu.VMEM((CHUNK,), jnp.float32),
            pltpu.VMEM((stripe,), jnp.float32),
            pltpu.VMEM_SHARED((n_tab,), jnp.float32),
            pltpu.VMEM_SHARED((n_tab,), jnp.float32),
        ],
    )
    def deg_kernel(src_hbm, dst_hbm, deg_out, srcidx, dstidx, ones_v, zstripe,
                   outdeg_sh, indeg_sh):
        c = lax.axis_index("c")
        s = lax.axis_index("s")
        w = c * NS + s

        _fill_1d(zstripe, stripe, 0.0)
        _fill_1d(ones_v, CHUNK, 1.0)
        pltpu.sync_copy(zstripe, outdeg_sh.at[pl.ds(s * stripe, stripe)])
        pltpu.sync_copy(zstripe, indeg_sh.at[pl.ds(s * stripe, stripe)])
        plsc.subcore_barrier()

        pltpu.sync_copy(src_hbm.at[w], srcidx)
        pltpu.sync_copy(dst_hbm.at[w], dstidx)

        def body(j, carry):
            pltpu.sync_copy(ones_v, outdeg_sh.at[srcidx.at[j]], add=True)
            pltpu.sync_copy(ones_v, indeg_sh.at[dstidx.at[j]], add=True)
            return carry

        lax.fori_loop(0, spw, body, 0)
        plsc.subcore_barrier()

        pltpu.sync_copy(outdeg_sh.at[pl.ds(s * stripe, stripe)],
                        deg_out.at[c, 0, pl.ds(s * stripe, stripe)])
        pltpu.sync_copy(indeg_sh.at[pl.ds(s * stripe, stripe)],
                        deg_out.at[c, 1, pl.ds(s * stripe, stripe)])

    return deg_kernel


def _make_scatter_kernel(n_acc, d, spw):
    mesh = plsc.VectorSubcoreMesh(core_axis_name="c", subcore_axis_name="s",
                                  num_cores=NC, num_subcores=NS)
    stripe = n_acc // NS
    assert stripe % CHUNK == 0 and stripe % 8 == 0

    @functools.partial(
        pl.kernel,
        out_type=jax.ShapeDtypeStruct((NC, n_acc, d), jnp.float32),
        mesh=mesh,
        scratch_types=[
            pltpu.VMEM((spw, CHUNK), jnp.int32),
            pltpu.VMEM((spw, CHUNK), jnp.int32),
            pltpu.VMEM((CHUNK, d), jnp.float32),
            pltpu.VMEM_SHARED((n_acc, d), jnp.float32),
            pltpu.SemaphoreType.DMA,
        ],
    )
    def scatter_kernel(src_hbm, dst_hbm, feat_hbm, s_out, srcidx, dstidx,
                       rows, s_sh, gsem):
        c = lax.axis_index("c")
        s = lax.axis_index("s")
        w = c * NS + s

        _zero_fill_2d(rows, CHUNK, d)
        for k in range(stripe // CHUNK):
            pltpu.sync_copy(
                rows, s_sh.at[pl.ds(s * stripe + k * CHUNK, CHUNK)])
        plsc.subcore_barrier()

        pltpu.sync_copy(src_hbm.at[w], srcidx)
        pltpu.sync_copy(dst_hbm.at[w], dstidx)

        def body(j, carry):
            pltpu.async_copy(feat_hbm.at[srcidx.at[j]], rows, gsem).wait()
            pltpu.sync_copy(rows, s_sh.at[dstidx.at[j]], add=True)
            return carry

        lax.fori_loop(0, spw, body, 0)
        plsc.subcore_barrier()

        pltpu.sync_copy(s_sh.at[pl.ds(s * stripe, stripe)],
                        s_out.at[c, pl.ds(s * stripe, stripe)])

    return scatter_kernel


def _make_prescale_body(n):
    def _prescale_body(feat_ref, deg_ref, out_ref, norm_ref):
        od = deg_ref[0, 0, :] + deg_ref[1, 0, :]
        out_ref[pl.ds(0, n), :] = (
            feat_ref[...] * lax.rsqrt(jnp.maximum(od, 1.0))[:, None])
        idg = deg_ref[0, 1, :] + deg_ref[1, 1, :]
        norm_ref[...] = lax.rsqrt(jnp.maximum(idg, 1.0))[:, None]
    return _prescale_body


def _epilogue_body(sp_ref, feat_ref, x_ref, norm_ref, ws_ref, wi_ref, out_ref):
    s_sum = sp_ref[0] + sp_ref[1]
    r = jnp.dot(s_sum, ws_ref[...], preferred_element_type=jnp.float32)
    r = r + jnp.dot(feat_ref[...] * s_sum, wi_ref[...],
                    preferred_element_type=jnp.float32)
    out_ref[...] = (r * norm_ref[...] + x_ref[...]) * 0.5


def kernel(feature, edge_index, weight_self, weight_interaction):
    n, d = feature.shape
    e = edge_index.shape[1]

    spw = -(-e // (NW * CHUNK))
    e_pad = NW * spw * CHUNK
    n_tab = NS * 128 * (-(-(n + 1) // (NS * 128)))
    n_acc = NS * 128 * (-(-(n + 1) // (NS * 128)))
    n_feat = 8 * (-(-(n + 1) // 8))

    pad = jnp.full((e_pad - e,), n, jnp.int32)
    src3d = jnp.concatenate([edge_index[0], pad]).reshape(NW, spw, CHUNK)
    dst3d = jnp.concatenate([edge_index[1], pad]).reshape(NW, spw, CHUNK)

    deg = _make_deg_kernel(n_tab, spw)(src3d, dst3d)
    deg = deg[:, :, :n]

    feat, norm_in = pl.pallas_call(
        _make_prescale_body(n),
        out_shape=(jax.ShapeDtypeStruct((n_feat, d), jnp.float32),
                   jax.ShapeDtypeStruct((n, 1), jnp.float32)),
    )(feature, deg)

    s_part = _make_scatter_kernel(n_acc, d, spw)(src3d, dst3d, feat)

    blk = 1000
    grid = n // blk
    rst = pl.pallas_call(
        _epilogue_body,
        grid=(grid,),
        in_specs=[
            pl.BlockSpec((NC, blk, d), lambda i: (0, i, 0)),
            pl.BlockSpec((blk, d), lambda i: (i, 0)),
            pl.BlockSpec((blk, d), lambda i: (i, 0)),
            pl.BlockSpec((blk, 1), lambda i: (i, 0)),
            pl.BlockSpec((d, d), lambda i: (0, 0)),
            pl.BlockSpec((d, d), lambda i: (0, 0)),
        ],
        out_specs=pl.BlockSpec((blk, d), lambda i: (i, 0)),
        out_shape=jax.ShapeDtypeStruct((n, d), jnp.float32),
    )(s_part, feat, feature, norm_in, weight_self, weight_interaction)

    return rst
